# Initial kernel scaffold; baseline (speedup 1.0000x reference)
import jax
import jax.numpy as jnp
from jax.experimental import pallas as pl


def kernel(x, edge_index, W1l, b1l, W1r, W2l, b2l, W2r, Wlin, blin):
    raise NotImplementedError("write your pallas kernel here")


# speedup vs baseline: 11.8193x; 11.8193x over previous
import functools

import jax
import jax.numpy as jnp
from jax import lax
from jax.experimental import pallas as pl
from jax.experimental.pallas import tpu as pltpu
from jax.experimental.pallas import tpu_sc as plsc

N = 10000
D = 128
H = 32

NC = 2
NS = 16
L = 16
NW = NC * NS

CHUNK = 128
N_PAD = 10240
ZR = N_PAD // NS


def _sc_mesh():
    return plsc.VectorSubcoreMesh(core_axis_name="c", subcore_axis_name="s")


_SC_PARAMS = pltpu.CompilerParams(use_tc_tiling_on_sc=False)


def _make_sc_segsum_wide(K):
    @functools.partial(
        pl.kernel,
        out_type=(
            jax.ShapeDtypeStruct((NC, N_PAD, H), jnp.float32),
            jax.ShapeDtypeStruct((NC, N_PAD), jnp.float32),
        ),
        mesh=_sc_mesh(),
        scratch_types=[
            pltpu.VMEM((K, CHUNK), jnp.int32),
            pltpu.VMEM((K, CHUNK), jnp.int32),
            pltpu.VMEM((CHUNK, H), jnp.float32),
            pltpu.VMEM((CHUNK,), jnp.float32),
            pltpu.VMEM_SHARED((N_PAD, H), jnp.float32),
            pltpu.VMEM_SHARED((N_PAD,), jnp.float32),
            pltpu.SemaphoreType.DMA,
        ],
        compiler_params=_SC_PARAMS,
    )
    def sc1(t1_hbm, srcs_hbm, dsts_hbm, z2_hbm, z1_hbm, sum_out, cnt_out,
            src_v, dst_v, rows_v, ones_v, acc_sh, cnt_sh, sem):
        cid = lax.axis_index("c")
        sid = lax.axis_index("s")
        wid = cid * NS + sid
        r0 = sid * ZR
        pltpu.sync_copy(srcs_hbm.at[wid], src_v)
        pltpu.sync_copy(dsts_hbm.at[wid], dst_v)
        pltpu.sync_copy(z2_hbm.at[pl.ds(r0, ZR)], acc_sh.at[pl.ds(r0, ZR)])
        pltpu.sync_copy(z1_hbm.at[pl.ds(r0, ZR)], cnt_sh.at[pl.ds(r0, ZR)])
        for i in range(CHUNK // L):
            ones_v[pl.ds(i * L, L)] = jnp.ones((L,), jnp.float32)
        plsc.subcore_barrier()

        def body(j, carry):
            pltpu.async_copy(t1_hbm.at[src_v.at[j]], rows_v, sem).wait()
            pltpu.sync_copy(rows_v, acc_sh.at[dst_v.at[j]], add=True)
            pltpu.sync_copy(ones_v, cnt_sh.at[dst_v.at[j]], add=True)
            return carry

        lax.fori_loop(0, K, body, 0)
        plsc.subcore_barrier()
        pltpu.sync_copy(acc_sh.at[pl.ds(r0, ZR)], sum_out.at[cid, pl.ds(r0, ZR)])
        pltpu.sync_copy(cnt_sh.at[pl.ds(r0, ZR)], cnt_out.at[cid, pl.ds(r0, ZR)])

    return sc1


def _make_sc_segsum_scalar(K):
    @functools.partial(
        pl.kernel,
        out_type=jax.ShapeDtypeStruct((NC, N_PAD), jnp.float32),
        mesh=_sc_mesh(),
        scratch_types=[
            pltpu.VMEM((K, CHUNK), jnp.int32),
            pltpu.VMEM((K, CHUNK), jnp.int32),
            pltpu.VMEM((CHUNK,), jnp.float32),
            pltpu.VMEM_SHARED((N_PAD,), jnp.float32),
            pltpu.SemaphoreType.DMA,
        ],
        compiler_params=_SC_PARAMS,
    )
    def sc2(p_hbm, srcs_hbm, dsts_hbm, z1_hbm, sum_out,
            src_v, dst_v, vals_v, acc_sh, sem):
        cid = lax.axis_index("c")
        sid = lax.axis_index("s")
        wid = cid * NS + sid
        r0 = sid * ZR
        pltpu.sync_copy(srcs_hbm.at[wid], src_v)
        pltpu.sync_copy(dsts_hbm.at[wid], dst_v)
        pltpu.sync_copy(z1_hbm.at[pl.ds(r0, ZR)], acc_sh.at[pl.ds(r0, ZR)])
        plsc.subcore_barrier()

        def body(j, carry):
            pltpu.async_copy(p_hbm.at[src_v.at[j]], vals_v, sem).wait()
            pltpu.sync_copy(vals_v, acc_sh.at[dst_v.at[j]], add=True)
            return carry

        lax.fori_loop(0, K, body, 0)
        plsc.subcore_barrier()
        pltpu.sync_copy(acc_sh.at[pl.ds(r0, ZR)], sum_out.at[cid, pl.ds(r0, ZR)])

    return sc2


def _tc1_body(x_ref, wl_ref, wr_ref, t1_ref, r1_ref):
    xv = x_ref[...]
    dn = (((1,), (1,)), ((), ()))
    t1_ref[...] = lax.dot_general(xv, wl_ref[...], dn,
                                  preferred_element_type=jnp.float32)
    r1_ref[...] = lax.dot_general(xv, wr_ref[...], dn,
                                  preferred_element_type=jnp.float32)


def _tc2_body(sum_ref, cnt_ref, r1_ref, b_ref, u_ref, v_ref, c2_ref,
              p_ref, q_ref, ic_ref):
    cc = jnp.maximum(cnt_ref[0] + cnt_ref[1], 1.0)
    ic = 1.0 / cc
    s = sum_ref[0] + sum_ref[1]
    h = jnp.maximum(s * ic[:, None] + b_ref[...] + r1_ref[...], 0.0)
    p_ref[...] = jnp.sum(h * u_ref[...], axis=1)
    q_ref[...] = jnp.sum(h * v_ref[...], axis=1) + c2_ref[...]
    ic_ref[...] = ic


def _tc3_body(s2_ref, ic_ref, q_ref, o_ref):
    o_ref[...] = (s2_ref[0] + s2_ref[1]) * ic_ref[...] + q_ref[...]


def kernel(x, edge_index, W1l, b1l, W1r, W2l, b2l, W2r, Wlin, blin):
    E = edge_index.shape[1]
    K = -(-E // (NW * CHUNK))
    E_pad = NW * K * CHUNK

    src = edge_index[0]
    dst = edge_index[1]
    pad = E_pad - E
    src_p = jnp.concatenate([src, jnp.zeros((pad,), jnp.int32)])
    dst_p = jnp.concatenate([dst, jnp.full((pad,), N, jnp.int32)])
    srcs = src_p.reshape(NW, K, CHUNK)
    dsts = dst_p.reshape(NW, K, CHUNK)
    x_pad = jnp.pad(x, ((0, N_PAD - N), (0, 0)))
    u = Wlin @ W2l
    v = Wlin @ W2r
    c2 = Wlin @ b2l + blin
    z2 = jnp.zeros((N_PAD, H), jnp.float32)
    z1 = jnp.zeros((N_PAD,), jnp.float32)

    t1, r1 = pl.pallas_call(
        _tc1_body,
        out_shape=(
            jax.ShapeDtypeStruct((N_PAD, H), jnp.float32),
            jax.ShapeDtypeStruct((N_PAD, H), jnp.float32),
        ),
    )(x_pad, W1l, W1r)

    sum1, cnt = _make_sc_segsum_wide(K)(t1, srcs, dsts, z2, z1)

    p, q, ic = pl.pallas_call(
        _tc2_body,
        out_shape=(
            jax.ShapeDtypeStruct((N_PAD,), jnp.float32),
            jax.ShapeDtypeStruct((N_PAD,), jnp.float32),
            jax.ShapeDtypeStruct((N_PAD,), jnp.float32),
        ),
    )(sum1, cnt, r1, b1l.reshape(1, H), u, v, c2)

    sum2 = _make_sc_segsum_scalar(K)(p, srcs, dsts, z1)

    out_full = pl.pallas_call(
        _tc3_body,
        out_shape=jax.ShapeDtypeStruct((N_PAD,), jnp.float32),
    )(sum2, ic, q)
    return out_full[:N, None]


# speedup vs baseline: 15.4777x; 1.3095x over previous
import functools

import jax
import jax.numpy as jnp
from jax import lax
from jax.experimental import pallas as pl
from jax.experimental.pallas import tpu as pltpu
from jax.experimental.pallas import tpu_sc as plsc

N = 10000
D = 128
H = 32

NC = 2
NS = 16
L = 16
NW = NC * NS

CHUNK = 128
N_PAD = 10240
ZR = N_PAD // NS


def _sc_mesh():
    return plsc.VectorSubcoreMesh(core_axis_name="c", subcore_axis_name="s")


_SC_PARAMS = pltpu.CompilerParams(use_tc_tiling_on_sc=False,
                                  needs_layout_passes=False)


def _make_sc_segsum_wide(K):
    @functools.partial(
        pl.kernel,
        out_type=(
            jax.ShapeDtypeStruct((NC, N_PAD, H), jnp.float32),
            jax.ShapeDtypeStruct((NC, N_PAD), jnp.float32),
        ),
        mesh=_sc_mesh(),
        scratch_types=[
            pltpu.VMEM((K, CHUNK), jnp.int32),
            pltpu.VMEM((K, CHUNK), jnp.int32),
            pltpu.VMEM((2, CHUNK, H), jnp.float32),
            pltpu.VMEM((CHUNK,), jnp.float32),
            pltpu.VMEM_SHARED((N_PAD, H), jnp.float32),
            pltpu.VMEM_SHARED((N_PAD,), jnp.float32),
            pltpu.SemaphoreType.DMA((2,)),
        ],
        compiler_params=_SC_PARAMS,
    )
    def sc1(t1_hbm, srcs_hbm, dsts_hbm, z2_hbm, z1_hbm, sum_out, cnt_out,
            src_v, dst_v, rows_v, ones_v, acc_sh, cnt_sh, sem):
        cid = lax.axis_index("c")
        sid = lax.axis_index("s")
        wid = cid * NS + sid
        r0 = sid * ZR
        pltpu.sync_copy(srcs_hbm.at[wid], src_v)
        pltpu.sync_copy(dsts_hbm.at[wid], dst_v)
        pltpu.async_copy(t1_hbm.at[src_v.at[0]], rows_v.at[0], sem.at[0])
        pltpu.sync_copy(z2_hbm.at[pl.ds(r0, ZR)], acc_sh.at[pl.ds(r0, ZR)])
        pltpu.sync_copy(z1_hbm.at[pl.ds(r0, ZR)], cnt_sh.at[pl.ds(r0, ZR)])
        for i in range(CHUNK // L):
            ones_v[pl.ds(i * L, L)] = jnp.ones((L,), jnp.float32)
        plsc.subcore_barrier()

        def body(g2, carry):
            for b in range(2):
                j = g2 * 2 + b
                nxt = j + 1

                @pl.when(nxt < K)
                def _():
                    pltpu.async_copy(t1_hbm.at[src_v.at[nxt]],
                                     rows_v.at[1 - b], sem.at[1 - b])

                pltpu.make_async_copy(t1_hbm.at[src_v.at[j]],
                                      rows_v.at[b], sem.at[b]).wait()
                pltpu.sync_copy(rows_v.at[b], acc_sh.at[dst_v.at[j]], add=True)
                pltpu.sync_copy(ones_v, cnt_sh.at[dst_v.at[j]], add=True)
            return carry

        lax.fori_loop(0, K // 2, body, 0)
        plsc.subcore_barrier()
        pltpu.sync_copy(acc_sh.at[pl.ds(r0, ZR)], sum_out.at[cid, pl.ds(r0, ZR)])
        pltpu.sync_copy(cnt_sh.at[pl.ds(r0, ZR)], cnt_out.at[cid, pl.ds(r0, ZR)])

    return sc1


def _make_sc_segsum_scalar(K):
    @functools.partial(
        pl.kernel,
        out_type=jax.ShapeDtypeStruct((NC, N_PAD), jnp.float32),
        mesh=_sc_mesh(),
        scratch_types=[
            pltpu.VMEM((K, CHUNK), jnp.int32),
            pltpu.VMEM((K, CHUNK), jnp.int32),
            pltpu.VMEM((N_PAD,), jnp.float32),
            pltpu.VMEM((K, CHUNK), jnp.float32),
            pltpu.VMEM_SHARED((N_PAD,), jnp.float32),
            pltpu.SemaphoreType.DMA,
        ],
        compiler_params=_SC_PARAMS,
    )
    def sc2(p_hbm, srcs_hbm, dsts_hbm, z1_hbm, sum_out,
            src_v, dst_v, p_v, vals_v, acc_sh, sem):
        cid = lax.axis_index("c")
        sid = lax.axis_index("s")
        wid = cid * NS + sid
        r0 = sid * ZR
        pltpu.sync_copy(srcs_hbm.at[wid], src_v)
        pltpu.sync_copy(dsts_hbm.at[wid], dst_v)
        pltpu.sync_copy(p_hbm, p_v)
        pltpu.sync_copy(z1_hbm.at[pl.ds(r0, ZR)], acc_sh.at[pl.ds(r0, ZR)])

        def gbody(j, carry):
            for c in range(CHUNK // L):
                idx = src_v[j, pl.ds(c * L, L)]
                vals_v[j, pl.ds(c * L, L)] = plsc.load_gather(p_v, [idx])
            return carry

        lax.fori_loop(0, K, gbody, 0)
        plsc.subcore_barrier()

        G = 8

        def sbody(g, carry):
            descs = [
                pltpu.async_copy(vals_v.at[g * G + b],
                                 acc_sh.at[dst_v.at[g * G + b]], sem, add=True)
                for b in range(G)
            ]
            for d in descs:
                d.wait()
            return carry

        lax.fori_loop(0, K // G, sbody, 0)
        plsc.subcore_barrier()
        pltpu.sync_copy(acc_sh.at[pl.ds(r0, ZR)], sum_out.at[cid, pl.ds(r0, ZR)])

    return sc2


def _tc1_body(x_ref, wl_ref, wr_ref, t1_ref, r1_ref):
    xv = x_ref[...]
    dn = (((1,), (1,)), ((), ()))
    t1_ref[...] = lax.dot_general(xv, wl_ref[...], dn,
                                  preferred_element_type=jnp.float32)
    r1_ref[...] = lax.dot_general(xv, wr_ref[...], dn,
                                  preferred_element_type=jnp.float32)


def _tc2_body(sum_ref, cnt_ref, r1_ref, b_ref, u_ref, v_ref, c2_ref,
              p_ref, q_ref, ic_ref):
    cc = jnp.maximum(cnt_ref[0] + cnt_ref[1], 1.0)
    ic = 1.0 / cc
    s = sum_ref[0] + sum_ref[1]
    h = jnp.maximum(s * ic[:, None] + b_ref[...] + r1_ref[...], 0.0)
    p_ref[...] = jnp.sum(h * u_ref[...], axis=1)
    q_ref[...] = jnp.sum(h * v_ref[...], axis=1) + c2_ref[...]
    ic_ref[...] = ic


def _tc3_body(s2_ref, ic_ref, q_ref, o_ref):
    o_ref[...] = (s2_ref[0] + s2_ref[1]) * ic_ref[...] + q_ref[...]


def kernel(x, edge_index, W1l, b1l, W1r, W2l, b2l, W2r, Wlin, blin):
    E = edge_index.shape[1]
    K = 8 * -(-E // (NW * CHUNK * 8))
    E_pad = NW * K * CHUNK

    src = edge_index[0]
    dst = edge_index[1]
    pad = E_pad - E
    src_p = jnp.concatenate([src, jnp.zeros((pad,), jnp.int32)])
    dst_p = jnp.concatenate([dst, jnp.full((pad,), N, jnp.int32)])
    srcs = src_p.reshape(NW, K, CHUNK)
    dsts = dst_p.reshape(NW, K, CHUNK)
    x_pad = jnp.pad(x, ((0, N_PAD - N), (0, 0)))
    u = Wlin @ W2l
    v = Wlin @ W2r
    c2 = Wlin @ b2l + blin
    z2 = jnp.zeros((N_PAD, H), jnp.float32)
    z1 = jnp.zeros((N_PAD,), jnp.float32)

    t1, r1 = pl.pallas_call(
        _tc1_body,
        out_shape=(
            jax.ShapeDtypeStruct((N_PAD, H), jnp.float32),
            jax.ShapeDtypeStruct((N_PAD, H), jnp.float32),
        ),
    )(x_pad, W1l, W1r)

    sum1, cnt = _make_sc_segsum_wide(K)(t1, srcs, dsts, z2, z1)

    p, q, ic = pl.pallas_call(
        _tc2_body,
        out_shape=(
            jax.ShapeDtypeStruct((N_PAD,), jnp.float32),
            jax.ShapeDtypeStruct((N_PAD,), jnp.float32),
            jax.ShapeDtypeStruct((N_PAD,), jnp.float32),
        ),
    )(sum1, cnt, r1, b1l.reshape(1, H), u, v, c2)

    sum2 = _make_sc_segsum_scalar(K)(p, srcs, dsts, z1)

    out_full = pl.pallas_call(
        _tc3_body,
        out_shape=jax.ShapeDtypeStruct((N_PAD,), jnp.float32),
    )(sum2, ic, q)
    return out_full[:N, None]
